# baseline (speedup 1.0000x reference)
import functools

import jax
import jax.numpy as jnp
from jax import lax
from jax.experimental import pallas as pl
from jax.experimental.pallas import tpu as pltpu
from jax.experimental.pallas import tpu_sc as plsc

Q = 1024
D = 32
DP = 128
N = 100000
K_TOP_K = 10
BLK = 2000
NBLK = N // BLK

_NC = 2
_NS = 16
_NW = _NC * _NS
_B_PER_W = Q // _NW


@functools.lru_cache(maxsize=1)
def _make_sc_gather():
  mesh = plsc.VectorSubcoreMesh(
      core_axis_name="c", subcore_axis_name="s", num_cores=_NC)

  @functools.partial(
      pl.kernel,
      mesh=mesh,
      out_type=jax.ShapeDtypeStruct((Q, D), jnp.float32),
      scratch_types=[
          pltpu.VMEM((_B_PER_W,), jnp.int32),
          pltpu.VMEM((_B_PER_W, D), jnp.float32),
          pltpu.SemaphoreType.DMA,
      ],
      compiler_params=pltpu.CompilerParams(use_tc_tiling_on_sc=False),
  )
  def sc_gather(table_hbm, idx_hbm, out_hbm, idx_v, rows_v, sem):
    wid = lax.axis_index("s") * _NC + lax.axis_index("c")
    base = wid * _B_PER_W
    pltpu.sync_copy(idx_hbm.at[pl.ds(base, _B_PER_W)], idx_v)
    pltpu.async_copy(table_hbm.at[idx_v], rows_v, sem).wait()
    pltpu.sync_copy(rows_v, out_hbm.at[pl.ds(base, _B_PER_W)])

  return sc_gather


def _count_body(m1_ref, gath_ref, m2p_ref, out_ref, t_ref, acc_ref):
  i = pl.program_id(0)

  @pl.when(i == 0)
  def _init():
    tmat = lax.dot_general(
        gath_ref[...], m1_ref[...], (((1,), (1,)), ((), ())),
        preferred_element_type=jnp.float32)
    r = lax.broadcasted_iota(jnp.int32, (Q, Q), 0)
    c = lax.broadcasted_iota(jnp.int32, (Q, Q), 1)
    tq = jnp.sum(jnp.where(r == c, tmat, 0.0), axis=0, keepdims=True)
    t_ref[...] = jnp.broadcast_to(tq, (8, Q))
    acc_ref[...] = jnp.zeros_like(acc_ref)

  scores = lax.dot_general(
      m2p_ref[:, 0:D], m1_ref[...], (((1,), (1,)), ((), ())),
      preferred_element_type=jnp.float32)
  hits = (scores.reshape(BLK // 8, 8, Q) > t_ref[...][None]).astype(jnp.int32)
  acc_ref[...] += jnp.sum(hits, axis=0)

  @pl.when(i == NBLK - 1)
  def _fin():
    cnt = jnp.sum(acc_ref[...], axis=0, keepdims=True)
    succ = (cnt < K_TOP_K).astype(jnp.float32)
    out_ref[0, 0] = jnp.sum(succ) / jnp.float32(Q)


_tc_count = pl.pallas_call(
    _count_body,
    grid=(NBLK,),
    in_specs=[
        pl.BlockSpec((Q, D), lambda i: (0, 0)),
        pl.BlockSpec((Q, D), lambda i: (0, 0)),
        pl.BlockSpec((BLK, DP), lambda i: (i, 0)),
    ],
    out_specs=pl.BlockSpec(
        (1, 1), lambda i: (0, 0), memory_space=pltpu.SMEM),
    out_shape=jax.ShapeDtypeStruct((1, 1), jnp.float32),
    scratch_shapes=[
        pltpu.VMEM((8, Q), jnp.float32),
        pltpu.VMEM((8, Q), jnp.int32),
    ],
    compiler_params=pltpu.CompilerParams(
        dimension_semantics=("arbitrary",)),
)


def kernel(modality1_features, modality2_features, groundtruth_all_indices):
  g = groundtruth_all_indices.astype(jnp.int32)
  m2p = jnp.pad(modality2_features, ((0, 0), (0, DP - D)))
  gath = _make_sc_gather()(modality2_features, g.reshape(Q))
  out = _tc_count(modality1_features, gath, m2p)
  return out[0, 0]


# speedup vs baseline: 1.3152x; 1.3152x over previous
import functools

import jax
import jax.numpy as jnp
from jax import lax
from jax.experimental import pallas as pl
from jax.experimental.pallas import tpu as pltpu
from jax.experimental.pallas import tpu_sc as plsc

Q = 1024
D = 32
DP = 128
N = 100000
K_TOP_K = 10
BLK = 4000
NBLK = N // BLK

_NC = 2
_NS = 16
_NW = _NC * _NS
_B_PER_W = Q // _NW


@functools.lru_cache(maxsize=1)
def _make_sc_gather():
  mesh = plsc.VectorSubcoreMesh(
      core_axis_name="c", subcore_axis_name="s", num_cores=_NC)

  @functools.partial(
      pl.kernel,
      mesh=mesh,
      out_type=jax.ShapeDtypeStruct((Q, DP), jnp.float32),
      scratch_types=[
          pltpu.VMEM((_B_PER_W,), jnp.int32),
          pltpu.VMEM((_B_PER_W, DP), jnp.float32),
          pltpu.SemaphoreType.DMA,
      ],
  )
  def sc_gather(table_hbm, idx_hbm, out_hbm, idx_v, rows_v, sem):
    wid = lax.axis_index("s") * _NC + lax.axis_index("c")
    base = wid * _B_PER_W
    pltpu.sync_copy(idx_hbm.at[pl.ds(base, _B_PER_W)], idx_v)
    pltpu.async_copy(table_hbm.at[idx_v], rows_v, sem).wait()
    pltpu.sync_copy(rows_v, out_hbm.at[pl.ds(base, _B_PER_W)])

  return sc_gather


def _count_body(m1_ref, gath_ref, m2p_ref, out_ref, t_ref, acc_ref):
  i = pl.program_id(0)

  @pl.when(i == 0)
  def _init():
    tmat = lax.dot_general(
        gath_ref[:, 0:D], m1_ref[...], (((1,), (1,)), ((), ())),
        preferred_element_type=jnp.float32)
    r = lax.broadcasted_iota(jnp.int32, (Q, Q), 0)
    c = lax.broadcasted_iota(jnp.int32, (Q, Q), 1)
    tq = jnp.sum(jnp.where(r == c, tmat, 0.0), axis=0, keepdims=True)
    t_ref[...] = jnp.broadcast_to(tq, (8, Q))
    acc_ref[...] = jnp.zeros_like(acc_ref)

  scores = lax.dot_general(
      m2p_ref[:, 0:D], m1_ref[...], (((1,), (1,)), ((), ())),
      preferred_element_type=jnp.float32)
  hits = (scores.reshape(BLK // 8, 8, Q) > t_ref[...][None]).astype(jnp.int32)
  acc_ref[...] += jnp.sum(hits, axis=0)

  @pl.when(i == NBLK - 1)
  def _fin():
    cnt = jnp.sum(acc_ref[...], axis=0, keepdims=True)
    succ = (cnt < K_TOP_K).astype(jnp.float32)
    out_ref[0, 0] = jnp.sum(succ) / jnp.float32(Q)


_tc_count = pl.pallas_call(
    _count_body,
    grid=(NBLK,),
    in_specs=[
        pl.BlockSpec((Q, D), lambda i: (0, 0)),
        pl.BlockSpec((Q, DP), lambda i: (0, 0)),
        pl.BlockSpec((BLK, DP), lambda i: (i, 0)),
    ],
    out_specs=pl.BlockSpec(
        (1, 1), lambda i: (0, 0), memory_space=pltpu.SMEM),
    out_shape=jax.ShapeDtypeStruct((1, 1), jnp.float32),
    scratch_shapes=[
        pltpu.VMEM((8, Q), jnp.float32),
        pltpu.VMEM((8, Q), jnp.int32),
    ],
    compiler_params=pltpu.CompilerParams(
        dimension_semantics=("arbitrary",)),
)


def kernel(modality1_features, modality2_features, groundtruth_all_indices):
  g = groundtruth_all_indices.astype(jnp.int32)
  m2p = jnp.pad(modality2_features, ((0, 0), (0, DP - D)))
  gath = _make_sc_gather()(m2p, g.reshape(Q))
  out = _tc_count(modality1_features, gath, m2p)
  return out[0, 0]


# speedup vs baseline: 1.3223x; 1.0054x over previous
import functools

import jax
import jax.numpy as jnp
from jax import lax
from jax.experimental import pallas as pl
from jax.experimental.pallas import tpu as pltpu
from jax.experimental.pallas import tpu_sc as plsc

Q = 1024
D = 32
DP = 128
N = 100000
K_TOP_K = 10
BLK = 5000
NBLK = N // BLK

_NC = 2
_NS = 16
_NW = _NC * _NS
_B_PER_W = Q // _NW


@functools.lru_cache(maxsize=1)
def _make_sc_gather():
  mesh = plsc.VectorSubcoreMesh(
      core_axis_name="c", subcore_axis_name="s", num_cores=_NC)

  @functools.partial(
      pl.kernel,
      mesh=mesh,
      out_type=jax.ShapeDtypeStruct((Q, DP), jnp.float32),
      scratch_types=[
          pltpu.VMEM((_B_PER_W,), jnp.int32),
          pltpu.VMEM((_B_PER_W, DP), jnp.float32),
          pltpu.SemaphoreType.DMA,
      ],
  )
  def sc_gather(table_hbm, idx_hbm, out_hbm, idx_v, rows_v, sem):
    wid = lax.axis_index("s") * _NC + lax.axis_index("c")
    base = wid * _B_PER_W
    pltpu.sync_copy(idx_hbm.at[pl.ds(base, _B_PER_W)], idx_v)
    pltpu.async_copy(table_hbm.at[idx_v], rows_v, sem).wait()
    pltpu.sync_copy(rows_v, out_hbm.at[pl.ds(base, _B_PER_W)])

  return sc_gather


def _count_body(m1_ref, gath_ref, m2p_ref, out_ref, t_ref, acc_ref):
  i = pl.program_id(0)

  @pl.when(i == 0)
  def _init():
    tmat = lax.dot_general(
        gath_ref[:, 0:D], m1_ref[...], (((1,), (1,)), ((), ())),
        preferred_element_type=jnp.float32)
    r = lax.broadcasted_iota(jnp.int32, (Q, Q), 0)
    c = lax.broadcasted_iota(jnp.int32, (Q, Q), 1)
    tq = jnp.sum(jnp.where(r == c, tmat, 0.0), axis=0, keepdims=True)
    t_ref[...] = jnp.broadcast_to(tq, (8, Q))
    acc_ref[...] = jnp.zeros_like(acc_ref)

  scores = lax.dot_general(
      m2p_ref[:, 0:D], m1_ref[...], (((1,), (1,)), ((), ())),
      preferred_element_type=jnp.float32)
  hits = (scores.reshape(BLK // 8, 8, Q) > t_ref[...][None]).astype(jnp.int32)
  acc_ref[...] += jnp.sum(hits, axis=0)

  @pl.when(i == NBLK - 1)
  def _fin():
    cnt = jnp.sum(acc_ref[...], axis=0, keepdims=True)
    succ = (cnt < K_TOP_K).astype(jnp.float32)
    out_ref[0, 0] = jnp.sum(succ) / jnp.float32(Q)


_tc_count = pl.pallas_call(
    _count_body,
    grid=(NBLK,),
    in_specs=[
        pl.BlockSpec((Q, D), lambda i: (0, 0)),
        pl.BlockSpec((Q, DP), lambda i: (0, 0)),
        pl.BlockSpec((BLK, DP), lambda i: (i, 0)),
    ],
    out_specs=pl.BlockSpec(
        (1, 1), lambda i: (0, 0), memory_space=pltpu.SMEM),
    out_shape=jax.ShapeDtypeStruct((1, 1), jnp.float32),
    scratch_shapes=[
        pltpu.VMEM((8, Q), jnp.float32),
        pltpu.VMEM((8, Q), jnp.int32),
    ],
    compiler_params=pltpu.CompilerParams(
        dimension_semantics=("arbitrary",)),
)


def kernel(modality1_features, modality2_features, groundtruth_all_indices):
  g = groundtruth_all_indices.astype(jnp.int32)
  m2p = jnp.pad(modality2_features, ((0, 0), (0, DP - D)))
  gath = _make_sc_gather()(m2p, g.reshape(Q))
  out = _tc_count(modality1_features, gath, m2p)
  return out[0, 0]
